# Initial kernel scaffold; baseline (speedup 1.0000x reference)
import jax
import jax.numpy as jnp
from jax.experimental import pallas as pl


def kernel(x, edge_index, Wl1, bl1, Wr1, Wl2, bl2, Wr2):
    raise NotImplementedError("write your pallas kernel here")


# speedup vs baseline: 5.7341x; 5.7341x over previous
import functools

import jax
import jax.numpy as jnp
from jax import lax
from jax.experimental import pallas as pl
from jax.experimental.pallas import tpu as pltpu
from jax.experimental.pallas import tpu_sc as plsc

_N = 10000
_F = 128
_E = 320000
_NC = 2
_NS = 16
_NW = _NC * _NS
_EPW = _E // _NW
_CHUNK = 128
_FULL = _EPW // _CHUNK
_TAIL = _EPW - _FULL * _CHUNK
_RPS = 624
_RT = _N - _NS * _RPS

_MESH = plsc.VectorSubcoreMesh(core_axis_name="c", subcore_axis_name="s")


def _init_acc(zrow_h, acc_s, s):
    row0 = s * _RPS
    pltpu.sync_copy(zrow_h.at[pl.ds(row0, _RPS)],
                    acc_s.at[pl.ds(row0, _RPS)])

    @pl.when(s == 0)
    def _tail():
        pltpu.sync_copy(zrow_h.at[pl.ds(_NS * _RPS, _RT)],
                        acc_s.at[pl.ds(_NS * _RPS, _RT)])


def _drain_acc(acc_s, out_h, c, s):
    row0 = s * _RPS
    pltpu.sync_copy(acc_s.at[pl.ds(row0, _RPS)],
                    out_h.at[c, pl.ds(row0, _RPS)])

    @pl.when(s == 0)
    def _tail():
        pltpu.sync_copy(acc_s.at[pl.ds(_NS * _RPS, _RT)],
                        out_h.at[c, pl.ds(_NS * _RPS, _RT)])


def _sc_aggregate(table, src, dst, zrow):
    @functools.partial(
        pl.kernel,
        out_type=jax.ShapeDtypeStruct((_NC, _N, _F), jnp.float32),
        mesh=_MESH,
        scratch_types=[
            pltpu.VMEM((_CHUNK,), jnp.int32),
            pltpu.VMEM((_CHUNK,), jnp.int32),
            pltpu.VMEM((_TAIL,), jnp.int32),
            pltpu.VMEM((_TAIL,), jnp.int32),
            pltpu.VMEM((_CHUNK, _F), jnp.float32),
            pltpu.VMEM((_TAIL, _F), jnp.float32),
            pltpu.VMEM_SHARED((_N, _F), jnp.float32),
            pltpu.SemaphoreType.DMA,
        ],
    )
    def k(table_h, src_h, dst_h, zrow_h, out_h,
          si_v, di_v, sit_v, dit_v, rows_v, rowst_v, acc_s, sem):
        c = lax.axis_index("c")
        s = lax.axis_index("s")
        wid = s * _NC + c
        _init_acc(zrow_h, acc_s, s)
        plsc.subcore_barrier()

        base = wid * _EPW

        @pl.loop(0, _FULL)
        def _(j):
            off = base + j * _CHUNK
            pltpu.sync_copy(src_h.at[pl.ds(off, _CHUNK)], si_v)
            pltpu.sync_copy(dst_h.at[pl.ds(off, _CHUNK)], di_v)
            pltpu.async_copy(table_h.at[si_v], rows_v, sem).wait()
            pltpu.sync_copy(rows_v, acc_s.at[di_v], add=True)

        offt = base + _FULL * _CHUNK
        pltpu.sync_copy(src_h.at[pl.ds(offt, _TAIL)], sit_v)
        pltpu.sync_copy(dst_h.at[pl.ds(offt, _TAIL)], dit_v)
        pltpu.async_copy(table_h.at[sit_v], rowst_v, sem).wait()
        pltpu.sync_copy(rowst_v, acc_s.at[dit_v], add=True)

        plsc.subcore_barrier()
        _drain_acc(acc_s, out_h, c, s)

    return k(table, src, dst, zrow)


def _sc_counts(dst, zrow, ones):
    @functools.partial(
        pl.kernel,
        out_type=jax.ShapeDtypeStruct((_NC, _N, _F), jnp.float32),
        mesh=_MESH,
        scratch_types=[
            pltpu.VMEM((_CHUNK,), jnp.int32),
            pltpu.VMEM((_TAIL,), jnp.int32),
            pltpu.VMEM((_CHUNK, _F), jnp.float32),
            pltpu.VMEM_SHARED((_N, _F), jnp.float32),
            pltpu.SemaphoreType.DMA,
        ],
    )
    def k(dst_h, zrow_h, ones_h, out_h, di_v, dit_v, ones_v, acc_s, sem):
        c = lax.axis_index("c")
        s = lax.axis_index("s")
        wid = s * _NC + c
        _init_acc(zrow_h, acc_s, s)
        pltpu.sync_copy(ones_h, ones_v)
        plsc.subcore_barrier()

        base = wid * _EPW

        @pl.loop(0, _FULL)
        def _(j):
            off = base + j * _CHUNK
            pltpu.sync_copy(dst_h.at[pl.ds(off, _CHUNK)], di_v)
            pltpu.sync_copy(ones_v, acc_s.at[di_v], add=True)

        offt = base + _FULL * _CHUNK
        pltpu.sync_copy(dst_h.at[pl.ds(offt, _TAIL)], dit_v)
        pltpu.sync_copy(ones_v.at[pl.ds(0, _TAIL)], acc_s.at[dit_v], add=True)

        plsc.subcore_barrier()
        _drain_acc(acc_s, out_h, c, s)

    return k(dst, zrow, ones)


def _tc_layer(partials, cntp, x, wl_t, bias, wr_t, relu):
    rows = 1000

    def body(p_ref, c_ref, x_ref, wl_ref, b_ref, wr_ref, o_ref):
        ssum = p_ref[0] + p_ref[1]
        cnt = c_ref[0, :, 0:1] + c_ref[1, :, 0:1]
        mean = ssum * (1.0 / jnp.maximum(cnt, 1.0))
        acc = jnp.dot(mean, wl_ref[...], preferred_element_type=jnp.float32,
                      precision=lax.Precision.HIGHEST)
        acc = acc + jnp.dot(x_ref[...], wr_ref[...],
                            preferred_element_type=jnp.float32,
                            precision=lax.Precision.HIGHEST)
        acc = acc + b_ref[...]
        if relu:
            acc = jnp.maximum(acc, 0.0)
        o_ref[...] = acc

    return pl.pallas_call(
        body,
        grid=(_N // rows,),
        in_specs=[
            pl.BlockSpec((_NC, rows, _F), lambda i: (0, i, 0)),
            pl.BlockSpec((_NC, rows, _F), lambda i: (0, i, 0)),
            pl.BlockSpec((rows, _F), lambda i: (i, 0)),
            pl.BlockSpec((_F, _F), lambda i: (0, 0)),
            pl.BlockSpec((1, _F), lambda i: (0, 0)),
            pl.BlockSpec((_F, _F), lambda i: (0, 0)),
        ],
        out_specs=pl.BlockSpec((rows, _F), lambda i: (i, 0)),
        out_shape=jax.ShapeDtypeStruct((_N, _F), jnp.float32),
    )(partials, cntp, x, wl_t, bias, wr_t)


def kernel(x, edge_index, Wl1, bl1, Wr1, Wl2, bl2, Wr2):
    src = edge_index[0]
    dst = edge_index[1]
    zrow = jnp.zeros((_N, _F), jnp.float32)
    ones = jnp.ones((_CHUNK, _F), jnp.float32)

    cntp = _sc_counts(dst, zrow, ones)
    p1 = _sc_aggregate(x, src, dst, zrow)
    h = _tc_layer(p1, cntp, x, Wl1.T, bl1.reshape(1, _F), Wr1.T, relu=True)
    p2 = _sc_aggregate(h, src, dst, zrow)
    out = _tc_layer(p2, cntp, h, Wl2.T, bl2.reshape(1, _F), Wr2.T, relu=False)
    return (out, out)


# speedup vs baseline: 7.9798x; 1.3916x over previous
import functools

import jax
import jax.numpy as jnp
from jax import lax
from jax.experimental import pallas as pl
from jax.experimental.pallas import tpu as pltpu
from jax.experimental.pallas import tpu_sc as plsc

_N = 10000
_F = 128
_E = 320000
_NC = 2
_NS = 16
_NW = _NC * _NS
_EPW = _E // _NW
_CHUNK = 128
_FULL = _EPW // _CHUNK
_TAIL = _EPW - _FULL * _CHUNK
_RPS = 624
_RT = _N - _NS * _RPS

_MESH = plsc.VectorSubcoreMesh(core_axis_name="c", subcore_axis_name="s")


def _init_acc(zrow_h, acc_s, s):
    row0 = s * _RPS
    pltpu.sync_copy(zrow_h.at[pl.ds(row0, _RPS)],
                    acc_s.at[pl.ds(row0, _RPS)])

    @pl.when(s == 0)
    def _tail():
        pltpu.sync_copy(zrow_h.at[pl.ds(_NS * _RPS, _RT)],
                        acc_s.at[pl.ds(_NS * _RPS, _RT)])


def _drain_acc(acc_s, out_h, c, s):
    row0 = s * _RPS
    pltpu.sync_copy(acc_s.at[pl.ds(row0, _RPS)],
                    out_h.at[c, pl.ds(row0, _RPS)])

    @pl.when(s == 0)
    def _tail():
        pltpu.sync_copy(acc_s.at[pl.ds(_NS * _RPS, _RT)],
                        out_h.at[c, pl.ds(_NS * _RPS, _RT)])


def _sc_aggregate(table, src, dst, zrow):
    @functools.partial(
        pl.kernel,
        out_type=jax.ShapeDtypeStruct((_NC, _N, _F), jnp.float32),
        mesh=_MESH,
        scratch_types=[
            pltpu.VMEM((_CHUNK,), jnp.int32),
            pltpu.VMEM((_CHUNK,), jnp.int32),
            pltpu.VMEM((_CHUNK,), jnp.int32),
            pltpu.VMEM((_CHUNK,), jnp.int32),
            pltpu.VMEM((_TAIL,), jnp.int32),
            pltpu.VMEM((_TAIL,), jnp.int32),
            pltpu.VMEM((_CHUNK, _F), jnp.float32),
            pltpu.VMEM((_CHUNK, _F), jnp.float32),
            pltpu.VMEM((_TAIL, _F), jnp.float32),
            pltpu.VMEM_SHARED((_N, _F), jnp.float32),
            pltpu.SemaphoreType.DMA,
            pltpu.SemaphoreType.DMA,
        ],
    )
    def k(table_h, src_h, dst_h, zrow_h, out_h,
          si0, si1, di0, di1, sit_v, dit_v, rows0, rows1, rowst_v,
          acc_s, sem0, sem1):
        c = lax.axis_index("c")
        s = lax.axis_index("s")
        wid = s * _NC + c
        base = wid * _EPW

        def load_idx(j, si_v, di_v):
            off = base + j * _CHUNK
            pltpu.sync_copy(src_h.at[pl.ds(off, _CHUNK)], si_v)
            pltpu.sync_copy(dst_h.at[pl.ds(off, _CHUNK)], di_v)

        load_idx(0, si0, di0)
        pltpu.async_copy(table_h.at[si0], rows0, sem0)
        _init_acc(zrow_h, acc_s, s)
        plsc.subcore_barrier()

        @pl.loop(0, _FULL // 2)
        def _(i):
            j1 = 2 * i + 1
            load_idx(j1, si1, di1)
            pltpu.async_copy(table_h.at[si1], rows1, sem1)
            pltpu.make_async_copy(table_h.at[si0], rows0, sem0).wait()
            pltpu.sync_copy(rows0, acc_s.at[di0], add=True)

            @pl.when(j1 + 1 < _FULL)
            def _prefetch():
                load_idx(j1 + 1, si0, di0)
                pltpu.async_copy(table_h.at[si0], rows0, sem0)

            pltpu.make_async_copy(table_h.at[si1], rows1, sem1).wait()
            pltpu.sync_copy(rows1, acc_s.at[di1], add=True)

        offt = base + _FULL * _CHUNK
        pltpu.sync_copy(src_h.at[pl.ds(offt, _TAIL)], sit_v)
        pltpu.sync_copy(dst_h.at[pl.ds(offt, _TAIL)], dit_v)
        pltpu.async_copy(table_h.at[sit_v], rowst_v, sem0).wait()
        pltpu.sync_copy(rowst_v, acc_s.at[dit_v], add=True)

        plsc.subcore_barrier()
        _drain_acc(acc_s, out_h, c, s)

    return k(table, src, dst, zrow)


def _sc_counts(dst, zrow, ones):
    @functools.partial(
        pl.kernel,
        out_type=jax.ShapeDtypeStruct((_NC, _N, _F), jnp.float32),
        mesh=_MESH,
        scratch_types=[
            pltpu.VMEM((_CHUNK,), jnp.int32),
            pltpu.VMEM((_TAIL,), jnp.int32),
            pltpu.VMEM((_CHUNK, _F), jnp.float32),
            pltpu.VMEM_SHARED((_N, _F), jnp.float32),
            pltpu.SemaphoreType.DMA,
        ],
    )
    def k(dst_h, zrow_h, ones_h, out_h, di_v, dit_v, ones_v, acc_s, sem):
        c = lax.axis_index("c")
        s = lax.axis_index("s")
        wid = s * _NC + c
        _init_acc(zrow_h, acc_s, s)
        pltpu.sync_copy(ones_h, ones_v)
        plsc.subcore_barrier()

        base = wid * _EPW

        @pl.loop(0, _FULL)
        def _(j):
            off = base + j * _CHUNK
            pltpu.sync_copy(dst_h.at[pl.ds(off, _CHUNK)], di_v)
            pltpu.sync_copy(ones_v, acc_s.at[di_v], add=True)

        offt = base + _FULL * _CHUNK
        pltpu.sync_copy(dst_h.at[pl.ds(offt, _TAIL)], dit_v)
        pltpu.sync_copy(ones_v.at[pl.ds(0, _TAIL)], acc_s.at[dit_v], add=True)

        plsc.subcore_barrier()
        _drain_acc(acc_s, out_h, c, s)

    return k(dst, zrow, ones)


def _tc_layer(partials, cntp, x, wl_t, bias, wr_t, relu):
    rows = 1000

    def body(p_ref, c_ref, x_ref, wl_ref, b_ref, wr_ref, o_ref):
        ssum = p_ref[0] + p_ref[1]
        cnt = c_ref[0, :, 0:1] + c_ref[1, :, 0:1]
        mean = ssum * (1.0 / jnp.maximum(cnt, 1.0))
        acc = jnp.dot(mean, wl_ref[...], preferred_element_type=jnp.float32,
                      precision=lax.Precision.HIGHEST)
        acc = acc + jnp.dot(x_ref[...], wr_ref[...],
                            preferred_element_type=jnp.float32,
                            precision=lax.Precision.HIGHEST)
        acc = acc + b_ref[...]
        if relu:
            acc = jnp.maximum(acc, 0.0)
        o_ref[...] = acc

    return pl.pallas_call(
        body,
        grid=(_N // rows,),
        in_specs=[
            pl.BlockSpec((_NC, rows, _F), lambda i: (0, i, 0)),
            pl.BlockSpec((_NC, rows, _F), lambda i: (0, i, 0)),
            pl.BlockSpec((rows, _F), lambda i: (i, 0)),
            pl.BlockSpec((_F, _F), lambda i: (0, 0)),
            pl.BlockSpec((1, _F), lambda i: (0, 0)),
            pl.BlockSpec((_F, _F), lambda i: (0, 0)),
        ],
        out_specs=pl.BlockSpec((rows, _F), lambda i: (i, 0)),
        out_shape=jax.ShapeDtypeStruct((_N, _F), jnp.float32),
    )(partials, cntp, x, wl_t, bias, wr_t)


def kernel(x, edge_index, Wl1, bl1, Wr1, Wl2, bl2, Wr2):
    src = edge_index[0]
    dst = edge_index[1]
    zrow = jnp.zeros((_N, _F), jnp.float32)
    ones = jnp.ones((_CHUNK, _F), jnp.float32)

    cntp = _sc_counts(dst, zrow, ones)
    p1 = _sc_aggregate(x, src, dst, zrow)
    h = _tc_layer(p1, cntp, x, Wl1.T, bl1.reshape(1, _F), Wr1.T, relu=True)
    p2 = _sc_aggregate(h, src, dst, zrow)
    out = _tc_layer(p2, cntp, h, Wl2.T, bl2.reshape(1, _F), Wr2.T, relu=False)
    return (out, out)


# speedup vs baseline: 9.9452x; 1.2463x over previous
import dataclasses
import functools

import jax
import jax.numpy as jnp
from jax import lax
from jax.experimental import pallas as pl
from jax.experimental.pallas import tpu as pltpu
from jax.experimental.pallas import tpu_sc as plsc

_N = 10000
_F = 128
_E = 320000
_NC = 2
_NS = 16
_NW = _NC * _NS
_EPW = _E // _NW
_CHUNK = 128
_FULL = _EPW // _CHUNK
_TAIL = _EPW - _FULL * _CHUNK
_RPS = 624
_RT = _N - _NS * _RPS

_MESH = plsc.VectorSubcoreMesh(core_axis_name="c", subcore_axis_name="s")


def _init_acc(zrow_h, acc_s, s):
    row0 = s * _RPS
    pltpu.sync_copy(zrow_h.at[pl.ds(row0, _RPS)],
                    acc_s.at[pl.ds(row0, _RPS)])

    @pl.when(s == 0)
    def _tail():
        pltpu.sync_copy(zrow_h.at[pl.ds(_NS * _RPS, _RT)],
                        acc_s.at[pl.ds(_NS * _RPS, _RT)])


def _drain_acc(acc_s, out_h, c, s):
    row0 = s * _RPS
    pltpu.sync_copy(acc_s.at[pl.ds(row0, _RPS)],
                    out_h.at[c, pl.ds(row0, _RPS)])

    @pl.when(s == 0)
    def _tail():
        pltpu.sync_copy(acc_s.at[pl.ds(_NS * _RPS, _RT)],
                        out_h.at[c, pl.ds(_NS * _RPS, _RT)])


def _sc_aggregate(table, src, dst, zrow, with_hist):
    out_type = [jax.ShapeDtypeStruct((_NC, _N, _F), jnp.float32)]
    if with_hist:
        out_type.append(jax.ShapeDtypeStruct((_NC, _NS, _N), jnp.float32))
    hist_scratch = [pltpu.VMEM((_N,), jnp.float32)] if with_hist else []
    cp = pltpu.CompilerParams()
    if with_hist and (
            "needs_layout_passes" in pltpu.CompilerParams.__dataclass_fields__):
        cp = dataclasses.replace(cp, needs_layout_passes=False)

    @functools.partial(
        pl.kernel,
        out_type=out_type,
        mesh=_MESH,
        compiler_params=cp,
        scratch_types=hist_scratch + [
            pltpu.VMEM((_CHUNK,), jnp.int32),
            pltpu.VMEM((_CHUNK,), jnp.int32),
            pltpu.VMEM((_CHUNK,), jnp.int32),
            pltpu.VMEM((_CHUNK,), jnp.int32),
            pltpu.VMEM((_TAIL,), jnp.int32),
            pltpu.VMEM((_TAIL,), jnp.int32),
            pltpu.VMEM((_CHUNK, _F), jnp.float32),
            pltpu.VMEM((_CHUNK, _F), jnp.float32),
            pltpu.VMEM((_TAIL, _F), jnp.float32),
            pltpu.VMEM_SHARED((_N, _F), jnp.float32),
            pltpu.SemaphoreType.DMA,
            pltpu.SemaphoreType.DMA,
        ],
    )
    def k(*refs):
        if with_hist:
            (table_h, src_h, dst_h, zrow_h, out_h, cnt_h, hist_v,
             si0, si1, di0, di1, sit_v, dit_v, rows0, rows1, rowst_v,
             acc_s, sem0, sem1) = refs
        else:
            (table_h, src_h, dst_h, zrow_h, out_h,
             si0, si1, di0, di1, sit_v, dit_v, rows0, rows1, rowst_v,
             acc_s, sem0, sem1) = refs
        c = lax.axis_index("c")
        s = lax.axis_index("s")
        wid = s * _NC + c
        base = wid * _EPW
        ones16 = jnp.full((16,), 1.0, dtype=jnp.float32)

        def load_idx(j, si_v, di_v):
            off = base + j * _CHUNK
            pltpu.sync_copy(src_h.at[pl.ds(off, _CHUNK)], si_v)
            pltpu.sync_copy(dst_h.at[pl.ds(off, _CHUNK)], di_v)

        def bump_hist(di_v, n):
            if with_hist:
                for kk in range(n // 16):
                    idx = di_v[pl.ds(kk * 16, 16)]
                    plsc.addupdate_scatter(hist_v, [idx], ones16)

        load_idx(0, si0, di0)
        pltpu.async_copy(table_h.at[si0], rows0, sem0)
        _init_acc(zrow_h, acc_s, s)
        if with_hist:
            @pl.loop(0, _N // 16)
            def _(i):
                hist_v[pl.ds(i * 16, 16)] = jnp.zeros((16,), jnp.float32)
        plsc.subcore_barrier()

        @pl.loop(0, _FULL // 2)
        def _(i):
            j1 = 2 * i + 1
            load_idx(j1, si1, di1)
            pltpu.async_copy(table_h.at[si1], rows1, sem1)
            pltpu.make_async_copy(table_h.at[si0], rows0, sem0).wait()
            pltpu.sync_copy(rows0, acc_s.at[di0], add=True)
            bump_hist(di0, _CHUNK)

            @pl.when(j1 + 1 < _FULL)
            def _prefetch():
                load_idx(j1 + 1, si0, di0)
                pltpu.async_copy(table_h.at[si0], rows0, sem0)

            pltpu.make_async_copy(table_h.at[si1], rows1, sem1).wait()
            pltpu.sync_copy(rows1, acc_s.at[di1], add=True)
            bump_hist(di1, _CHUNK)

        offt = base + _FULL * _CHUNK
        pltpu.sync_copy(src_h.at[pl.ds(offt, _TAIL)], sit_v)
        pltpu.sync_copy(dst_h.at[pl.ds(offt, _TAIL)], dit_v)
        pltpu.async_copy(table_h.at[sit_v], rowst_v, sem0).wait()
        pltpu.sync_copy(rowst_v, acc_s.at[dit_v], add=True)
        bump_hist(dit_v, _TAIL)

        plsc.subcore_barrier()
        _drain_acc(acc_s, out_h, c, s)
        if with_hist:
            pltpu.sync_copy(hist_v, cnt_h.at[c, s])

    return k(table, src, dst, zrow)


def _tc_rinv(hists):
    def body(h_ref, o_ref):
        cnt = jnp.sum(h_ref[...], axis=(0, 1))
        o_ref[...] = (1.0 / jnp.maximum(cnt, 1.0))[None, :]

    return pl.pallas_call(
        body,
        out_shape=jax.ShapeDtypeStruct((1, _N), jnp.float32),
    )(hists)


def _tc_layer(partials, rinv, x, wl_t, bias, wr_t, relu):
    rows = 1000

    def body(p_ref, r_ref, x_ref, wl_ref, b_ref, wr_ref, o_ref):
        mean = (p_ref[0] + p_ref[1]) * r_ref[...]
        acc = jnp.dot(mean, wl_ref[...], preferred_element_type=jnp.float32,
                      precision=lax.Precision.HIGHEST)
        acc = acc + jnp.dot(x_ref[...], wr_ref[...],
                            preferred_element_type=jnp.float32,
                            precision=lax.Precision.HIGHEST)
        acc = acc + b_ref[...]
        if relu:
            acc = jnp.maximum(acc, 0.0)
        o_ref[...] = acc

    return pl.pallas_call(
        body,
        grid=(_N // rows,),
        in_specs=[
            pl.BlockSpec((_NC, rows, _F), lambda i: (0, i, 0)),
            pl.BlockSpec((rows, 1), lambda i: (i, 0)),
            pl.BlockSpec((rows, _F), lambda i: (i, 0)),
            pl.BlockSpec((_F, _F), lambda i: (0, 0)),
            pl.BlockSpec((1, _F), lambda i: (0, 0)),
            pl.BlockSpec((_F, _F), lambda i: (0, 0)),
        ],
        out_specs=pl.BlockSpec((rows, _F), lambda i: (i, 0)),
        out_shape=jax.ShapeDtypeStruct((_N, _F), jnp.float32),
    )(partials, rinv, x, wl_t, bias, wr_t)


def kernel(x, edge_index, Wl1, bl1, Wr1, Wl2, bl2, Wr2):
    src = edge_index[0]
    dst = edge_index[1]
    zrow = jnp.zeros((_N, _F), jnp.float32)

    p1, hists = _sc_aggregate(x, src, dst, zrow, with_hist=True)
    rinv = _tc_rinv(hists).reshape(_N, 1)
    h = _tc_layer(p1, rinv, x, Wl1.T, bl1.reshape(1, _F), Wr1.T, relu=True)
    (p2,) = _sc_aggregate(h, src, dst, zrow, with_hist=False)
    out = _tc_layer(p2, rinv, h, Wl2.T, bl2.reshape(1, _F), Wr2.T, relu=False)
    return (out, out)


# speedup vs baseline: 11.0724x; 1.1133x over previous
import dataclasses
import functools

import jax
import jax.numpy as jnp
from jax import lax
from jax.experimental import pallas as pl
from jax.experimental.pallas import tpu as pltpu
from jax.experimental.pallas import tpu_sc as plsc

_N = 10000
_F = 128
_E = 320000
_NC = 2
_NS = 16
_NW = _NC * _NS
_EPW = _E // _NW
_CHUNK = 128
_FULL = _EPW // _CHUNK
_TAIL = _EPW - _FULL * _CHUNK
_RPS = 624
_RT = _N - _NS * _RPS

_MESH = plsc.VectorSubcoreMesh(core_axis_name="c", subcore_axis_name="s")


def _init_acc(zrow_h, acc_s, s):
    row0 = s * _RPS
    pltpu.sync_copy(zrow_h.at[pl.ds(row0, _RPS)],
                    acc_s.at[pl.ds(row0, _RPS)])

    @pl.when(s == 0)
    def _tail():
        pltpu.sync_copy(zrow_h.at[pl.ds(_NS * _RPS, _RT)],
                        acc_s.at[pl.ds(_NS * _RPS, _RT)])


def _drain_acc(acc_s, out_h, c, s):
    row0 = s * _RPS
    pltpu.sync_copy(acc_s.at[pl.ds(row0, _RPS)],
                    out_h.at[c, pl.ds(row0, _RPS)])

    @pl.when(s == 0)
    def _tail():
        pltpu.sync_copy(acc_s.at[pl.ds(_NS * _RPS, _RT)],
                        out_h.at[c, pl.ds(_NS * _RPS, _RT)])


def _sc_aggregate(table, src, dst, zrow, with_hist):
    out_type = [jax.ShapeDtypeStruct((_NC, _N, _F), jnp.float32)]
    if with_hist:
        out_type.append(jax.ShapeDtypeStruct((_NC, _NS, _N), jnp.float32))
    hist_scratch = [pltpu.VMEM((_N,), jnp.float32)] if with_hist else []
    cp = pltpu.CompilerParams()
    if with_hist and (
            "needs_layout_passes" in pltpu.CompilerParams.__dataclass_fields__):
        cp = dataclasses.replace(cp, needs_layout_passes=False)

    @functools.partial(
        pl.kernel,
        out_type=out_type,
        mesh=_MESH,
        compiler_params=cp,
        scratch_types=hist_scratch + [
            pltpu.VMEM((_CHUNK,), jnp.int32),
            pltpu.VMEM((_CHUNK,), jnp.int32),
            pltpu.VMEM((_CHUNK,), jnp.int32),
            pltpu.VMEM((_CHUNK,), jnp.int32),
            pltpu.VMEM((_CHUNK,), jnp.int32),
            pltpu.VMEM((_CHUNK,), jnp.int32),
            pltpu.VMEM((_TAIL,), jnp.int32),
            pltpu.VMEM((_TAIL,), jnp.int32),
            pltpu.VMEM((_CHUNK, _F), jnp.float32),
            pltpu.VMEM((_CHUNK, _F), jnp.float32),
            pltpu.VMEM((_TAIL, _F), jnp.float32),
            pltpu.VMEM_SHARED((_N, _F), jnp.float32),
            pltpu.SemaphoreType.DMA,
            pltpu.SemaphoreType.DMA,
            pltpu.SemaphoreType.DMA,
            pltpu.SemaphoreType.DMA,
            pltpu.SemaphoreType.DMA,
            pltpu.SemaphoreType.DMA,
        ],
    )
    def k(*refs):
        if with_hist:
            (table_h, src_h, dst_h, zrow_h, out_h, cnt_h, hist_v,
             si0, si1, di0, di1, ds0, ds1, sit_v, dit_v,
             rows0, rows1, rowst_v, acc_s,
             semg0, semg1, sems0, sems1, semi0, semi1) = refs
        else:
            (table_h, src_h, dst_h, zrow_h, out_h,
             si0, si1, di0, di1, ds0, ds1, sit_v, dit_v,
             rows0, rows1, rowst_v, acc_s,
             semg0, semg1, sems0, sems1, semi0, semi1) = refs
        c = lax.axis_index("c")
        s = lax.axis_index("s")
        wid = s * _NC + c
        base = wid * _EPW
        ones16 = jnp.full((16,), 1.0, dtype=jnp.float32)
        rows = (rows0, rows1)
        sis = (si0, si1)
        dis = (di0, di1)
        dss = (ds0, ds1)
        semg = (semg0, semg1)
        sems = (sems0, sems1)
        semi = (semi0, semi1)

        def issue_idx(j, b):
            off = base + j * _CHUNK
            pltpu.async_copy(src_h.at[pl.ds(off, _CHUNK)], sis[b], semi[b])
            pltpu.async_copy(dst_h.at[pl.ds(off, _CHUNK)], dis[b], semi[b])

        def wait_idx(j, b):
            off = base + j * _CHUNK
            pltpu.make_async_copy(src_h.at[pl.ds(off, _CHUNK)], sis[b],
                                  semi[b]).wait()
            pltpu.make_async_copy(dst_h.at[pl.ds(off, _CHUNK)], dis[b],
                                  semi[b]).wait()

        def bump_hist(di_v, n):
            if with_hist:
                for kk in range(n // 16):
                    idx = di_v[pl.ds(kk * 16, 16)]
                    plsc.addupdate_scatter(hist_v, [idx], ones16)

        issue_idx(0, 0)
        issue_idx(1, 1)
        wait_idx(0, 0)
        pltpu.async_copy(table_h.at[si0], rows0, semg0)
        _init_acc(zrow_h, acc_s, s)
        if with_hist:
            @pl.loop(0, _N // 16)
            def _(i):
                hist_v[pl.ds(i * 16, 16)] = jnp.zeros((16,), jnp.float32)
        plsc.subcore_barrier()

        def step(j, b):
            pltpu.make_async_copy(table_h.at[sis[b]], rows[b],
                                  semg[b]).wait()
            for kk in range(_CHUNK // 16):
                dss[b][pl.ds(kk * 16, 16)] = dis[b][pl.ds(kk * 16, 16)]
            pltpu.async_copy(rows[b], acc_s.at[dss[b]], sems[b], add=True)
            bump_hist(dis[b], _CHUNK)

            @pl.when(j + 2 < _FULL)
            def _prefetch_idx():
                issue_idx(j + 2, b)

            jn = j + 1
            bn = 1 - b

            @pl.when(jn < _FULL)
            def _issue_gather():
                @pl.when(jn >= 2)
                def _drain_prev():
                    pltpu.make_async_copy(rows[bn], acc_s.at[dss[bn]],
                                          sems[bn]).wait()
                wait_idx(jn, bn)
                pltpu.async_copy(table_h.at[sis[bn]], rows[bn], semg[bn])

        @pl.loop(0, _FULL // 2)
        def _(i):
            step(2 * i, 0)
            step(2 * i + 1, 1)

        pltpu.make_async_copy(rows0, acc_s.at[ds0], sems0).wait()
        pltpu.make_async_copy(rows1, acc_s.at[ds1], sems1).wait()

        offt = base + _FULL * _CHUNK
        pltpu.sync_copy(src_h.at[pl.ds(offt, _TAIL)], sit_v)
        pltpu.sync_copy(dst_h.at[pl.ds(offt, _TAIL)], dit_v)
        pltpu.async_copy(table_h.at[sit_v], rowst_v, semg0).wait()
        pltpu.sync_copy(rowst_v, acc_s.at[dit_v], add=True)
        bump_hist(dit_v, _TAIL)

        plsc.subcore_barrier()
        _drain_acc(acc_s, out_h, c, s)
        if with_hist:
            pltpu.sync_copy(hist_v, cnt_h.at[c, s])

    return k(table, src, dst, zrow)


def _tc_rinv(hists):
    def body(h_ref, o_ref):
        cnt = jnp.sum(h_ref[...], axis=(0, 1))
        o_ref[...] = (1.0 / jnp.maximum(cnt, 1.0))[None, :]

    return pl.pallas_call(
        body,
        out_shape=jax.ShapeDtypeStruct((1, _N), jnp.float32),
    )(hists)


def _tc_layer(partials, rinv, x, wl_t, bias, wr_t, relu):
    rows = 1000

    def body(p_ref, r_ref, x_ref, wl_ref, b_ref, wr_ref, o_ref):
        mean = (p_ref[0] + p_ref[1]) * r_ref[...]
        acc = jnp.dot(mean, wl_ref[...], preferred_element_type=jnp.float32,
                      precision=lax.Precision.HIGHEST)
        acc = acc + jnp.dot(x_ref[...], wr_ref[...],
                            preferred_element_type=jnp.float32,
                            precision=lax.Precision.HIGHEST)
        acc = acc + b_ref[...]
        if relu:
            acc = jnp.maximum(acc, 0.0)
        o_ref[...] = acc

    return pl.pallas_call(
        body,
        grid=(_N // rows,),
        in_specs=[
            pl.BlockSpec((_NC, rows, _F), lambda i: (0, i, 0)),
            pl.BlockSpec((rows, 1), lambda i: (i, 0)),
            pl.BlockSpec((rows, _F), lambda i: (i, 0)),
            pl.BlockSpec((_F, _F), lambda i: (0, 0)),
            pl.BlockSpec((1, _F), lambda i: (0, 0)),
            pl.BlockSpec((_F, _F), lambda i: (0, 0)),
        ],
        out_specs=pl.BlockSpec((rows, _F), lambda i: (i, 0)),
        out_shape=jax.ShapeDtypeStruct((_N, _F), jnp.float32),
    )(partials, rinv, x, wl_t, bias, wr_t)


def kernel(x, edge_index, Wl1, bl1, Wr1, Wl2, bl2, Wr2):
    src = edge_index[0]
    dst = edge_index[1]
    zrow = jnp.zeros((_N, _F), jnp.float32)

    p1, hists = _sc_aggregate(x, src, dst, zrow, with_hist=True)
    rinv = _tc_rinv(hists).reshape(_N, 1)
    h = _tc_layer(p1, rinv, x, Wl1.T, bl1.reshape(1, _F), Wr1.T, relu=True)
    (p2,) = _sc_aggregate(h, src, dst, zrow, with_hist=False)
    out = _tc_layer(p2, rinv, h, Wl2.T, bl2.reshape(1, _F), Wr2.T, relu=False)
    return (out, out)
